# baseline (speedup 1.0000x reference)
import functools
import jax
import jax.numpy as jnp
from jax import lax
from jax.experimental import pallas as pl

H = 384
N_BLK = 1000
E_BLK = 2000
NEG_SLOPE = 0.2
PPR_ALPHA = 0.15


def _node_body(emb_ref, w_ref, asrc_ref, adst_ref, q_ref,
               h_ref, as_ref, ad_ref, sc_ref):
    emb = emb_ref[...]
    h = jnp.dot(emb, w_ref[...], preferred_element_type=jnp.float32)
    h_ref[...] = h
    as_ref[0, 0, :] = lax.dot_general(
        h, asrc_ref[...], (((1,), (1,)), ((), ())),
        preferred_element_type=jnp.float32)[:, 0]
    ad_ref[0, 0, :] = lax.dot_general(
        h, adst_ref[...], (((1,), (1,)), ((), ())),
        preferred_element_type=jnp.float32)[:, 0]
    sc_ref[0] = lax.dot_general(
        q_ref[...], emb, (((1,), (1,)), ((), ())),
        preferred_element_type=jnp.float32)


def _edge_body(attr_ref, ve_ref, w_ref, ea_ref):
    a = attr_ref[...]
    m = jnp.mean(a, axis=1)
    w_ref[0, 0, :] = jnp.maximum(m, 0.0) + jnp.log(1.0 + jnp.exp(-jnp.abs(m)))
    ea_ref[0, 0, :] = lax.dot_general(
        a, ve_ref[...], (((1,), (1,)), ((), ())),
        preferred_element_type=jnp.float32)[:, 0]


def kernel(query, edge_index, edge_attr, node_emb, W, W_edge,
           att_src, att_dst, att_edge, bias):
    row = edge_index[0]
    col = edge_index[1]
    n = node_emb.shape[0]
    b = query.shape[0]
    e = edge_attr.shape[0]
    n_blocks = n // N_BLK
    e_blocks = e // E_BLK

    h, a_src3, a_dst3, scores = pl.pallas_call(
        _node_body,
        grid=(n_blocks,),
        in_specs=[
            pl.BlockSpec((N_BLK, H), lambda i: (i, 0)),
            pl.BlockSpec((H, H), lambda i: (0, 0)),
            pl.BlockSpec((1, H), lambda i: (0, 0)),
            pl.BlockSpec((1, H), lambda i: (0, 0)),
            pl.BlockSpec((b, H), lambda i: (0, 0)),
        ],
        out_specs=[
            pl.BlockSpec((N_BLK, H), lambda i: (i, 0)),
            pl.BlockSpec((1, 1, N_BLK), lambda i: (i, 0, 0)),
            pl.BlockSpec((1, 1, N_BLK), lambda i: (i, 0, 0)),
            pl.BlockSpec((1, b, N_BLK), lambda i: (i, 0, 0)),
        ],
        out_shape=[
            jax.ShapeDtypeStruct((n, H), jnp.float32),
            jax.ShapeDtypeStruct((n_blocks, 1, N_BLK), jnp.float32),
            jax.ShapeDtypeStruct((n_blocks, 1, N_BLK), jnp.float32),
            jax.ShapeDtypeStruct((n_blocks, b, N_BLK), jnp.float32),
        ],
    )(node_emb, W, att_src.reshape(1, H), att_dst.reshape(1, H), query)
    a_src = a_src3.reshape(n)
    a_dst = a_dst3.reshape(n)
    scores = jnp.transpose(scores, (1, 0, 2)).reshape(b, n)

    v_e = W_edge @ att_edge
    w3, ea3 = pl.pallas_call(
        _edge_body,
        grid=(e_blocks,),
        in_specs=[
            pl.BlockSpec((E_BLK, H), lambda i: (i, 0)),
            pl.BlockSpec((1, H), lambda i: (0, 0)),
        ],
        out_specs=[
            pl.BlockSpec((1, 1, E_BLK), lambda i: (i, 0, 0)),
            pl.BlockSpec((1, 1, E_BLK), lambda i: (i, 0, 0)),
        ],
        out_shape=[
            jax.ShapeDtypeStruct((e_blocks, 1, E_BLK), jnp.float32),
            jax.ShapeDtypeStruct((e_blocks, 1, E_BLK), jnp.float32),
        ],
    )(edge_attr, v_e.reshape(1, H))
    w = w3.reshape(e)
    e_att = ea3.reshape(e)

    _, seed = jax.lax.top_k(scores, 5)
    row_sum = jnp.zeros((n,), jnp.float32).at[row].add(w)
    w_norm = w / (row_sum[row] + 1e-9)
    p0 = jnp.zeros((b, n), jnp.float32).at[jnp.arange(b)[:, None], seed].set(1.0)
    p = p0
    for _ in range(3):
        msgs = p[:, row] * w_norm[None, :]
        prop = jnp.zeros((b, n), jnp.float32).at[:, col].add(msgs)
        p = PPR_ALPHA * p0 + (1.0 - PPR_ALPHA) * prop

    logits = a_src[row] + a_dst[col] + e_att
    logits = jnp.maximum(logits, NEG_SLOPE * logits)
    ex = jnp.exp(logits)
    denom = jnp.zeros((n,), jnp.float32).at[col].add(ex)
    attn = ex / (denom[col] + 1e-9)
    node_feats = jnp.zeros((n, H), jnp.float32).at[col].add(attn[:, None] * h[row]) + bias
    out = p @ node_feats
    return out
